# baseline (speedup 1.0000x reference)
import functools

import jax
import jax.numpy as jnp
from jax import lax
from jax.experimental import pallas as pl
from jax.experimental.pallas import tpu as pltpu
from jax.experimental.pallas import tpu_sc as plsc

NC = 2
NS = 16
L = 16
G = 128
SLOTS = 5


def _build_sag(n, e, d):
    nt = NC * NS
    rpt = ((n + nt - 1) // nt + L - 1) // L * L
    n_pad = rpt * nt
    trash = NS * rpt
    acc_rows = ((NS * rpt + 1 + 7) // 8) * 8
    rp_cols = ((rpt + 1 + L - 1) // L) * L
    nv_rp = rp_cols // L
    nv_g = G // L

    mesh = plsc.VectorSubcoreMesh(core_axis_name="c", subcore_axis_name="s")

    @functools.partial(
        pl.kernel,
        mesh=mesh,
        out_type=jax.ShapeDtypeStruct((n_pad, d), jnp.float32),
        scratch_types=[
            pltpu.VMEM((rp_cols,), jnp.int32),
            pltpu.VMEM((SLOTS, G), jnp.int32),
            pltpu.VMEM((SLOTS, G), jnp.int32),
            pltpu.VMEM((G,), jnp.int32),
            pltpu.VMEM((SLOTS, G, d), jnp.float32),
            pltpu.VMEM((L, d), jnp.float32),
            pltpu.VMEM_SHARED((acc_rows, d), jnp.float32),
        ] + [pltpu.SemaphoreType.DMA] * (3 * SLOTS),
        compiler_params=pltpu.CompilerParams(needs_layout_passes=False),
    )
    def sag(x_hbm, rpt_hbm, col_hbm, out_hbm,
            rp_t, colbuf, idxbuf, cnt, gbuf, zbuf, acc, *sems):
        sem_c = sems[0:SLOTS]
        sem_g = sems[SLOTS:2 * SLOTS]
        sem_s = sems[2 * SLOTS:3 * SLOTS]
        cid = lax.axis_index("c")
        sid = lax.axis_index("s")
        wid = sid * NC + cid
        r0 = wid * rpt
        racc = sid * rpt

        pltpu.sync_copy(rpt_hbm.at[wid], rp_t)

        zero_f = jnp.zeros((L,), jnp.float32)
        for i in range(L):
            for j in range(d // L):
                zbuf[i, pl.ds(j * L, L)] = zero_f
        for i in range(rpt // L):
            pltpu.sync_copy(zbuf, acc.at[pl.ds(racc + i * L, L)])

        rp0 = rp_t[pl.ds(0, L)][0]
        rend = rp_t[pl.ds(rpt - rpt % L, L)][rpt % L]
        a = (rp0 // 8) * 8
        nch = (rend - a + G - 1) // G
        ngroups = (nch + SLOTS - 1) // SLOTS

        iota = lax.broadcasted_iota(jnp.int32, (L,), 0)
        ones_i = jnp.ones((L,), jnp.int32)
        zero_i = jnp.zeros((L,), jnp.int32)
        not_lane0 = iota >= 1

        def scatter_wait(b):
            pltpu.make_async_copy(gbuf.at[b], acc.at[idxbuf.at[b]], sem_s[b]).wait()

        for b in range(SLOTS):
            @pl.when(b < nch)
            def _():
                pltpu.async_copy(col_hbm.at[pl.ds(a + b * G, G)], colbuf.at[b],
                                 sem_c[b])

        def group(p, carry):
            ks = [SLOTS * p + b for b in range(SLOTS)]
            for b in range(SLOTS):
                k = ks[b]

                @pl.when(k >= SLOTS)
                def _():
                    scatter_wait(b)

                @pl.when(k < nch)
                def _(k=k, b=b):
                    pltpu.make_async_copy(col_hbm.at[pl.ds(a + k * G, G)],
                                          colbuf.at[b], sem_c[b]).wait()
                    pltpu.async_copy(x_hbm.at[colbuf.at[b]], gbuf.at[b], sem_g[b])

            cy = carry
            for b in range(SLOTS):
                k = ks[b]
                ebase = a + k * G
                for v in range(nv_g):
                    cnt[pl.ds(v * L, L)] = zero_i
                for v in range(nv_rp):
                    pos = rp_t[pl.ds(v * L, L)] - ebase
                    msk = (pos >= 0) & (pos < G)
                    if v == 0:
                        msk = msk & not_lane0
                    plsc.addupdate_scatter(cnt, [pos], ones_i, mask=msk)
                cyb = cy
                for v in range(nv_g):
                    run = plsc.cumsum(cnt[pl.ds(v * L, L)]) + cyb
                    epos = iota + (ebase + v * L)
                    keep = (epos >= rp0) & (epos < rend)
                    idxbuf[b, pl.ds(v * L, L)] = jnp.where(keep, racc + run, trash)
                    cyb = run[L - 1]
                cy = jnp.where(k < nch, cyb, cy)

            for b in range(SLOTS):
                k = ks[b]

                @pl.when(k < nch)
                def _(k=k, b=b):
                    pltpu.make_async_copy(x_hbm.at[colbuf.at[b]], gbuf.at[b],
                                          sem_g[b]).wait()

                    @pl.when(k + SLOTS < nch)
                    def _():
                        pltpu.async_copy(
                            col_hbm.at[pl.ds(a + (k + SLOTS) * G, G)],
                            colbuf.at[b], sem_c[b])

                    pltpu.async_copy(gbuf.at[b], acc.at[idxbuf.at[b]], sem_s[b],
                                     add=True)
            return cy

        lax.fori_loop(0, ngroups, group, jnp.int32(0))

        for b in range(SLOTS):
            j = SLOTS * (ngroups - 1) + b

            @pl.when((j >= 0) & (j < nch))
            def _(b=b):
                scatter_wait(b)

        pltpu.sync_copy(acc.at[pl.ds(racc, rpt)], out_hbm.at[pl.ds(r0, rpt)])

    return sag, rpt, n_pad, rp_cols


def kernel(X, row_pointers, column_index, blockPartition, edgeToColumn,
           edgeToRow, hybrid_type, row_nzr, col_nzr):
    n, d = X.shape
    e = column_index.shape[0]
    sag, rpt, n_pad, rp_cols = _build_sag(n, e, d)

    col_pad = jnp.concatenate(
        [column_index, jnp.zeros((SLOTS * G,), jnp.int32)])
    rp_ext = jnp.concatenate(
        [row_pointers.astype(jnp.int32),
         jnp.full((n_pad + rp_cols - (n + 1),), e, jnp.int32)])
    nt = NC * NS
    rp_tiles = rp_ext[jnp.arange(nt)[:, None] * rpt + jnp.arange(rp_cols)[None, :]]

    out = sag(X, rp_tiles, col_pad)
    return out[:n]


# speedup vs baseline: 1.0219x; 1.0219x over previous
import functools

import jax
import jax.numpy as jnp
from jax import lax
from jax.experimental import pallas as pl
from jax.experimental.pallas import tpu as pltpu
from jax.experimental.pallas import tpu_sc as plsc

NC = 2
NS = 16
L = 16
G = 128
SLOTS = 5


def _build_sag(n, e, d):
    nt = NC * NS
    rpt = ((n + nt - 1) // nt + L - 1) // L * L
    trash = NS * rpt
    acc_rows = ((NS * rpt + 1 + 7) // 8) * 8
    rp_cols = ((rpt + 1 + L - 1) // L) * L
    nv_rp = rp_cols // L
    nv_g = G // L
    emax = e - G
    assert n % 8 == 0 and e % 8 == 0 and n >= L

    mesh = plsc.VectorSubcoreMesh(core_axis_name="c", subcore_axis_name="s")

    @functools.partial(
        pl.kernel,
        mesh=mesh,
        out_type=jax.ShapeDtypeStruct((n, d), jnp.float32),
        scratch_types=[
            pltpu.VMEM((rp_cols,), jnp.int32),
            pltpu.VMEM((SLOTS, G), jnp.int32),
            pltpu.VMEM((SLOTS, G), jnp.int32),
            pltpu.VMEM((G,), jnp.int32),
            pltpu.VMEM((SLOTS, G, d), jnp.float32),
            pltpu.VMEM((L, d), jnp.float32),
            pltpu.VMEM_SHARED((acc_rows, d), jnp.float32),
        ] + [pltpu.SemaphoreType.DMA] * (3 * SLOTS + 1),
        compiler_params=pltpu.CompilerParams(needs_layout_passes=False),
    )
    def sag(x_hbm, rpt_hbm, col_hbm, out_hbm,
            rp_t, colbuf, idxbuf, cnt, gbuf, zbuf, acc, *sems):
        sem_c = sems[0:SLOTS]
        sem_g = sems[SLOTS:2 * SLOTS]
        sem_s = sems[2 * SLOTS:3 * SLOTS]
        sem_o = sems[3 * SLOTS]
        cid = lax.axis_index("c")
        sid = lax.axis_index("s")
        wid = sid * NC + cid
        r0 = wid * rpt
        racc = sid * rpt

        pltpu.sync_copy(rpt_hbm.at[wid], rp_t)

        zero_f = jnp.zeros((L,), jnp.float32)
        for i in range(L):
            for j in range(d // L):
                zbuf[i, pl.ds(j * L, L)] = zero_f
        for i in range(rpt // L):
            pltpu.sync_copy(zbuf, acc.at[pl.ds(racc + i * L, L)])

        rp0 = rp_t[pl.ds(0, L)][0]
        rend = rp_t[pl.ds(rpt - rpt % L, L)][rpt % L]
        a = (rp0 // 8) * 8
        nch = (rend - a + G - 1) // G
        ngroups = (nch + SLOTS - 1) // SLOTS

        iota = lax.broadcasted_iota(jnp.int32, (L,), 0)
        ones_i = jnp.ones((L,), jnp.int32)
        zero_i = jnp.zeros((L,), jnp.int32)
        not_lane0 = iota >= 1

        def rbase(k):
            return jnp.minimum(a + k * G, emax)

        def scatter_wait(b):
            pltpu.make_async_copy(gbuf.at[b], acc.at[idxbuf.at[b]], sem_s[b]).wait()

        for b in range(SLOTS):
            @pl.when(b < nch)
            def _(b=b):
                pltpu.async_copy(col_hbm.at[pl.ds(rbase(b), G)], colbuf.at[b],
                                 sem_c[b])

        def group(p, carry):
            ks = [SLOTS * p + b for b in range(SLOTS)]
            for b in range(SLOTS):
                k = ks[b]

                @pl.when(k >= SLOTS)
                def _():
                    scatter_wait(b)

                @pl.when(k < nch)
                def _(k=k, b=b):
                    pltpu.make_async_copy(col_hbm.at[pl.ds(rbase(k), G)],
                                          colbuf.at[b], sem_c[b]).wait()
                    pltpu.async_copy(x_hbm.at[colbuf.at[b]], gbuf.at[b], sem_g[b])

            cy = carry
            for b in range(SLOTS):
                k = ks[b]
                nominal = a + k * G
                ebase = jnp.minimum(nominal, emax)
                shift = nominal - ebase
                for v in range(nv_g):
                    cnt[pl.ds(v * L, L)] = zero_i
                for v in range(nv_rp):
                    pos = rp_t[pl.ds(v * L, L)] - ebase
                    msk = (pos >= shift) & (pos < G)
                    if v == 0:
                        msk = msk & not_lane0
                    plsc.addupdate_scatter(cnt, [pos], ones_i, mask=msk)
                cyb = cy
                for v in range(nv_g):
                    run = plsc.cumsum(cnt[pl.ds(v * L, L)]) + cyb
                    epos = iota + (ebase + v * L)
                    keep = (epos >= rp0) & (epos >= nominal) & (epos < rend)
                    idxbuf[b, pl.ds(v * L, L)] = jnp.where(keep, racc + run, trash)
                    cyb = run[L - 1]
                cy = jnp.where(k < nch, cyb, cy)

            for b in range(SLOTS):
                k = ks[b]

                @pl.when(k < nch)
                def _(k=k, b=b):
                    pltpu.make_async_copy(x_hbm.at[colbuf.at[b]], gbuf.at[b],
                                          sem_g[b]).wait()

                    @pl.when(k + SLOTS < nch)
                    def _():
                        pltpu.async_copy(col_hbm.at[pl.ds(rbase(k + SLOTS), G)],
                                         colbuf.at[b], sem_c[b])

                    pltpu.async_copy(gbuf.at[b], acc.at[idxbuf.at[b]], sem_s[b],
                                     add=True)
            return cy

        lax.fori_loop(0, ngroups, group, jnp.int32(0))

        for b in range(SLOTS):
            j = SLOTS * (ngroups - 1) + b

            @pl.when((j >= 0) & (j < nch))
            def _(b=b):
                scatter_wait(b)

        for i in range(rpt // L):
            dst = jnp.minimum(r0 + i * L, n - L)
            pltpu.async_copy(acc.at[pl.ds(racc + (dst - r0), L)],
                             out_hbm.at[pl.ds(dst, L)], sem_o)
        for i in range(rpt // L):
            pltpu.make_async_copy(acc.at[pl.ds(racc, L)],
                                  out_hbm.at[pl.ds(r0, L)], sem_o).wait()

    return sag, rpt, rp_cols


def kernel(X, row_pointers, column_index, blockPartition, edgeToColumn,
           edgeToRow, hybrid_type, row_nzr, col_nzr):
    n, d = X.shape
    e = column_index.shape[0]
    sag, rpt, rp_cols = _build_sag(n, e, d)

    nt = NC * NS
    rp_ext = jnp.concatenate(
        [row_pointers.astype(jnp.int32),
         jnp.full((nt * rpt + rp_cols - (n + 1),), e, jnp.int32)])
    rp_tiles = rp_ext[jnp.arange(nt)[:, None] * rpt + jnp.arange(rp_cols)[None, :]]

    return sag(X, rp_tiles, column_index)


# speedup vs baseline: 1.0400x; 1.0178x over previous
import functools

import jax
import jax.numpy as jnp
from jax import lax
from jax.experimental import pallas as pl
from jax.experimental.pallas import tpu as pltpu
from jax.experimental.pallas import tpu_sc as plsc

NC = 2
NS = 16
L = 16
G = 128
SLOTS = 5


def _build_sag(n, e, d):
    nt = NC * NS
    rpt = ((n + nt - 1) // nt + L - 1) // L * L
    trash = NS * rpt
    acc_rows = ((NS * rpt + 1 + 7) // 8) * 8
    rp_cols = ((rpt + 1 + L - 1) // L) * L
    nv_rp = rp_cols // L
    nv_g = G // L
    emax = e - G
    assert n % 8 == 0 and e % 8 == 0 and n >= L

    mesh = plsc.VectorSubcoreMesh(core_axis_name="c", subcore_axis_name="s")

    @functools.partial(
        pl.kernel,
        mesh=mesh,
        out_type=jax.ShapeDtypeStruct((n, d), jnp.float32),
        scratch_types=[
            pltpu.VMEM((rp_cols,), jnp.int32),
            pltpu.VMEM((SLOTS, G), jnp.int32),
            pltpu.VMEM((SLOTS, G), jnp.int32),
            pltpu.VMEM((G,), jnp.int32),
            pltpu.VMEM((SLOTS, G, d), jnp.float32),
            pltpu.VMEM((L, d), jnp.float32),
            pltpu.VMEM_SHARED((acc_rows, d), jnp.float32),
        ] + [pltpu.SemaphoreType.DMA] * (3 * SLOTS + 1),
        compiler_params=pltpu.CompilerParams(needs_layout_passes=False),
    )
    def sag(x_hbm, rpt_hbm, col_hbm, out_hbm,
            rp_t, colbuf, idxbuf, cnt, gbuf, zbuf, acc, *sems):
        sem_c = sems[0:SLOTS]
        sem_g = sems[SLOTS:2 * SLOTS]
        sem_s = sems[2 * SLOTS:3 * SLOTS]
        sem_o = sems[3 * SLOTS]
        cid = lax.axis_index("c")
        sid = lax.axis_index("s")
        wid = sid * NC + cid
        r0 = wid * rpt
        racc = sid * rpt

        pltpu.sync_copy(rpt_hbm.at[wid], rp_t)

        zero_f = jnp.zeros((L,), jnp.float32)
        for i in range(L):
            for j in range(d // L):
                zbuf[i, pl.ds(j * L, L)] = zero_f
        for i in range(rpt // L):
            pltpu.async_copy(zbuf, acc.at[pl.ds(racc + i * L, L)], sem_o)

        rp0 = rp_t[pl.ds(0, L)][0]
        rend = rp_t[pl.ds(rpt - rpt % L, L)][rpt % L]
        a = (rp0 // 8) * 8
        nch = (rend - a + G - 1) // G
        ngroups = (nch + SLOTS - 1) // SLOTS

        iota = lax.broadcasted_iota(jnp.int32, (L,), 0)
        ones_i = jnp.ones((L,), jnp.int32)
        zero_i = jnp.zeros((L,), jnp.int32)
        not_lane0 = iota >= 1

        def rbase(k):
            return jnp.minimum(a + k * G, emax)

        def scatter_wait(b):
            pltpu.make_async_copy(gbuf.at[b], acc.at[idxbuf.at[b]], sem_s[b]).wait()

        for b in range(SLOTS):
            @pl.when(b < nch)
            def _(b=b):
                pltpu.async_copy(col_hbm.at[pl.ds(rbase(b), G)], colbuf.at[b],
                                 sem_c[b])

        for i in range(rpt // L):
            pltpu.make_async_copy(zbuf, acc.at[pl.ds(racc, L)], sem_o).wait()

        def group(p, carry):
            ks = [SLOTS * p + b for b in range(SLOTS)]
            for b in range(SLOTS):
                k = ks[b]

                @pl.when(k >= SLOTS)
                def _():
                    scatter_wait(b)

                @pl.when(k < nch)
                def _(k=k, b=b):
                    pltpu.make_async_copy(col_hbm.at[pl.ds(rbase(k), G)],
                                          colbuf.at[b], sem_c[b]).wait()
                    pltpu.async_copy(x_hbm.at[colbuf.at[b]], gbuf.at[b], sem_g[b])

            cy = carry
            for b in range(SLOTS):
                k = ks[b]
                nominal = a + k * G
                ebase = jnp.minimum(nominal, emax)
                shift = nominal - ebase
                for v in range(nv_g):
                    cnt[pl.ds(v * L, L)] = zero_i
                for v in range(nv_rp):
                    pos = rp_t[pl.ds(v * L, L)] - ebase
                    msk = (pos >= shift) & (pos < G)
                    if v == 0:
                        msk = msk & not_lane0
                    plsc.addupdate_scatter(cnt, [pos], ones_i, mask=msk)
                cyb = cy
                for v in range(nv_g):
                    run = plsc.cumsum(cnt[pl.ds(v * L, L)]) + cyb
                    epos = iota + (ebase + v * L)
                    keep = (epos >= rp0) & (epos >= nominal) & (epos < rend)
                    idxbuf[b, pl.ds(v * L, L)] = jnp.where(keep, racc + run, trash)
                    cyb = run[L - 1]
                cy = jnp.where(k < nch, cyb, cy)

            for b in range(SLOTS):
                k = ks[b]

                @pl.when(k < nch)
                def _(k=k, b=b):
                    pltpu.make_async_copy(x_hbm.at[colbuf.at[b]], gbuf.at[b],
                                          sem_g[b]).wait()

                    @pl.when(k + SLOTS < nch)
                    def _():
                        pltpu.async_copy(col_hbm.at[pl.ds(rbase(k + SLOTS), G)],
                                         colbuf.at[b], sem_c[b])

                    pltpu.async_copy(gbuf.at[b], acc.at[idxbuf.at[b]], sem_s[b],
                                     add=True)
            return cy

        lax.fori_loop(0, ngroups, group, jnp.int32(0))

        for b in range(SLOTS):
            j = SLOTS * (ngroups - 1) + b

            @pl.when((j >= 0) & (j < nch))
            def _(b=b):
                scatter_wait(b)

        for i in range(rpt // L):
            dst = jnp.minimum(r0 + i * L, n - L)
            pltpu.async_copy(acc.at[pl.ds(racc + (dst - r0), L)],
                             out_hbm.at[pl.ds(dst, L)], sem_o)
        for i in range(rpt // L):
            pltpu.make_async_copy(acc.at[pl.ds(racc, L)],
                                  out_hbm.at[pl.ds(r0, L)], sem_o).wait()

    return sag, rpt, rp_cols


def kernel(X, row_pointers, column_index, blockPartition, edgeToColumn,
           edgeToRow, hybrid_type, row_nzr, col_nzr):
    n, d = X.shape
    e = column_index.shape[0]
    sag, rpt, rp_cols = _build_sag(n, e, d)

    nt = NC * NS
    rp_ext = jnp.concatenate(
        [row_pointers.astype(jnp.int32),
         jnp.full((nt * rpt + rp_cols - (n + 1),), e, jnp.int32)])
    rp_tiles = rp_ext[jnp.arange(nt)[:, None] * rpt + jnp.arange(rp_cols)[None, :]]

    return sag(X, rp_tiles, column_index)


# speedup vs baseline: 1.0938x; 1.0517x over previous
import functools

import jax
import jax.numpy as jnp
from jax import lax
from jax.experimental import pallas as pl
from jax.experimental.pallas import tpu as pltpu
from jax.experimental.pallas import tpu_sc as plsc

NC = 2
NS = 16
L = 16
G = 128
SLOTS = 5


def _build_sag(n, e, d):
    nt = NC * NS
    rpt = ((n + nt - 1) // nt + L - 1) // L * L
    trash = NS * rpt
    acc_rows = ((NS * rpt + 1 + 7) // 8) * 8
    rp_cols = ((rpt + 1 + L - 1) // L) * L
    nv_rp = rp_cols // L
    nv_g = G // L
    emax = e - G
    assert n % 8 == 0 and e % 8 == 0 and n >= L

    mesh = plsc.VectorSubcoreMesh(core_axis_name="c", subcore_axis_name="s")

    @functools.partial(
        pl.kernel,
        mesh=mesh,
        out_type=jax.ShapeDtypeStruct((n, d), jnp.float32),
        scratch_types=[
            pltpu.VMEM((rp_cols,), jnp.int32),
            pltpu.VMEM((SLOTS, G), jnp.int32),
            pltpu.VMEM((SLOTS, G), jnp.int32),
            pltpu.VMEM((G,), jnp.int32),
            pltpu.VMEM((SLOTS, G, d), jnp.float32),
            pltpu.VMEM((L, d), jnp.float32),
            pltpu.VMEM_SHARED((acc_rows, d), jnp.float32),
        ] + [pltpu.SemaphoreType.DMA] * (3 * SLOTS + 1),
        compiler_params=pltpu.CompilerParams(needs_layout_passes=False),
    )
    def sag(x_hbm, rpt_hbm, col_hbm, out_hbm,
            rp_t, colbuf, idxbuf, cnt, gbuf, zbuf, acc, *sems):
        sem_c = sems[0:SLOTS]
        sem_g = sems[SLOTS:2 * SLOTS]
        sem_s = sems[2 * SLOTS:3 * SLOTS]
        sem_o = sems[3 * SLOTS]
        cid = lax.axis_index("c")
        sid = lax.axis_index("s")
        wid = sid * NC + cid
        r0 = wid * rpt
        racc = sid * rpt

        pltpu.sync_copy(rpt_hbm.at[pl.ds(wid * rpt, rp_cols)], rp_t)

        zero_f = jnp.zeros((L,), jnp.float32)
        for i in range(L):
            for j in range(d // L):
                zbuf[i, pl.ds(j * L, L)] = zero_f
        for i in range(rpt // L):
            pltpu.async_copy(zbuf, acc.at[pl.ds(racc + i * L, L)], sem_o)

        rp0 = rp_t[pl.ds(0, L)][0]
        rend = rp_t[pl.ds(rpt - rpt % L, L)][rpt % L]
        a = (rp0 // 8) * 8
        nch = (rend - a + G - 1) // G
        ngroups = (nch + SLOTS - 1) // SLOTS

        iota = lax.broadcasted_iota(jnp.int32, (L,), 0)
        ones_i = jnp.ones((L,), jnp.int32)
        zero_i = jnp.zeros((L,), jnp.int32)
        not_lane0 = iota >= 1

        def rbase(k):
            return jnp.minimum(a + k * G, emax)

        def scatter_wait(b):
            pltpu.make_async_copy(gbuf.at[b], acc.at[idxbuf.at[b]], sem_s[b]).wait()

        for b in range(SLOTS):
            @pl.when(b < nch)
            def _(b=b):
                pltpu.async_copy(col_hbm.at[pl.ds(rbase(b), G)], colbuf.at[b],
                                 sem_c[b])

        for i in range(rpt // L):
            pltpu.make_async_copy(zbuf, acc.at[pl.ds(racc, L)], sem_o).wait()

        def group(p, carry):
            ks = [SLOTS * p + b for b in range(SLOTS)]
            for b in range(SLOTS):
                k = ks[b]

                @pl.when(k >= SLOTS)
                def _():
                    scatter_wait(b)

                @pl.when(k < nch)
                def _(k=k, b=b):
                    pltpu.make_async_copy(col_hbm.at[pl.ds(rbase(k), G)],
                                          colbuf.at[b], sem_c[b]).wait()
                    pltpu.async_copy(x_hbm.at[colbuf.at[b]], gbuf.at[b], sem_g[b])

            cy = carry
            for b in range(SLOTS):
                k = ks[b]
                nominal = a + k * G
                ebase = jnp.minimum(nominal, emax)
                shift = nominal - ebase
                for v in range(nv_g):
                    cnt[pl.ds(v * L, L)] = zero_i
                for v in range(nv_rp):
                    pos = rp_t[pl.ds(v * L, L)] - ebase
                    msk = (pos >= shift) & (pos < G)
                    if v == 0:
                        msk = msk & not_lane0
                    plsc.addupdate_scatter(cnt, [pos], ones_i, mask=msk)
                cyb = cy
                for v in range(nv_g):
                    run = plsc.cumsum(cnt[pl.ds(v * L, L)]) + cyb
                    epos = iota + (ebase + v * L)
                    keep = (epos >= rp0) & (epos >= nominal) & (epos < rend)
                    idxbuf[b, pl.ds(v * L, L)] = jnp.where(keep, racc + run, trash)
                    cyb = run[L - 1]
                cy = jnp.where(k < nch, cyb, cy)

            for b in range(SLOTS):
                k = ks[b]

                @pl.when(k < nch)
                def _(k=k, b=b):
                    pltpu.make_async_copy(x_hbm.at[colbuf.at[b]], gbuf.at[b],
                                          sem_g[b]).wait()

                    @pl.when(k + SLOTS < nch)
                    def _():
                        pltpu.async_copy(col_hbm.at[pl.ds(rbase(k + SLOTS), G)],
                                         colbuf.at[b], sem_c[b])

                    pltpu.async_copy(gbuf.at[b], acc.at[idxbuf.at[b]], sem_s[b],
                                     add=True)
            return cy

        lax.fori_loop(0, ngroups, group, jnp.int32(0))

        for b in range(SLOTS):
            j = SLOTS * (ngroups - 1) + b

            @pl.when((j >= 0) & (j < nch))
            def _(b=b):
                scatter_wait(b)

        for i in range(rpt // L):
            dst = jnp.minimum(r0 + i * L, n - L)
            pltpu.async_copy(acc.at[pl.ds(racc + (dst - r0), L)],
                             out_hbm.at[pl.ds(dst, L)], sem_o)
        for i in range(rpt // L):
            pltpu.make_async_copy(acc.at[pl.ds(racc, L)],
                                  out_hbm.at[pl.ds(r0, L)], sem_o).wait()

    return sag, rpt, rp_cols


def kernel(X, row_pointers, column_index, blockPartition, edgeToColumn,
           edgeToRow, hybrid_type, row_nzr, col_nzr):
    n, d = X.shape
    e = column_index.shape[0]
    sag, rpt, rp_cols = _build_sag(n, e, d)

    nt = NC * NS
    rp_ext = jnp.concatenate(
        [row_pointers.astype(jnp.int32),
         jnp.full((nt * rpt + rp_cols - (n + 1),), e, jnp.int32)])

    return sag(X, rp_ext, column_index)


# speedup vs baseline: 1.1044x; 1.0096x over previous
import functools

import jax
import jax.numpy as jnp
from jax import lax
from jax.experimental import pallas as pl
from jax.experimental.pallas import tpu as pltpu
from jax.experimental.pallas import tpu_sc as plsc

NC = 2
NS = 16
L = 16
G = 128
SLOTS = 5


def _build_sag(n, e, d):
    nt = NC * NS
    rpt = ((n + nt - 1) // nt + L - 1) // L * L
    trash = NS * rpt
    acc_rows = ((NS * rpt + 1 + 7) // 8) * 8
    rp_cols = ((rpt + 1 + L - 1) // L) * L
    nv_rp = rp_cols // L
    nv_g = G // L
    emax = e - G
    assert n % 8 == 0 and e % 8 == 0 and n >= L

    mesh = plsc.VectorSubcoreMesh(core_axis_name="c", subcore_axis_name="s")

    @functools.partial(
        pl.kernel,
        mesh=mesh,
        out_type=jax.ShapeDtypeStruct((n, d), jnp.float32),
        scratch_types=[
            pltpu.VMEM((rp_cols,), jnp.int32),
            pltpu.VMEM((SLOTS, G), jnp.int32),
            pltpu.VMEM((SLOTS, G), jnp.int32),
            pltpu.VMEM((G,), jnp.int32),
            pltpu.VMEM((SLOTS, G, d), jnp.float32),
            pltpu.VMEM((L, d), jnp.float32),
            pltpu.VMEM_SHARED((acc_rows, d), jnp.float32),
        ] + [pltpu.SemaphoreType.DMA] * (3 * SLOTS + 1),
        compiler_params=pltpu.CompilerParams(needs_layout_passes=False),
    )
    def sag(x_hbm, rpt_hbm, col_hbm, out_hbm,
            rp_t, colbuf, idxbuf, cnt, gbuf, zbuf, acc, *sems):
        sem_c = sems[0:SLOTS]
        sem_g = sems[SLOTS:2 * SLOTS]
        sem_s = sems[2 * SLOTS:3 * SLOTS]
        sem_o = sems[3 * SLOTS]
        cid = lax.axis_index("c")
        sid = lax.axis_index("s")
        wid = sid * NC + cid
        r0 = wid * rpt
        racc = sid * rpt

        pltpu.sync_copy(rpt_hbm.at[pl.ds(wid * rpt, rp_cols)], rp_t)

        zero_f = jnp.zeros((L,), jnp.float32)
        for i in range(L):
            for j in range(d // L):
                zbuf[i, pl.ds(j * L, L)] = zero_f
        for i in range(rpt // L):
            pltpu.async_copy(zbuf, acc.at[pl.ds(racc + i * L, L)], sem_o)

        rp0 = rp_t[pl.ds(0, L)][0]
        rend = rp_t[pl.ds(rpt - rpt % L, L)][rpt % L]
        a = (rp0 // 8) * 8
        nch = (rend - a + G - 1) // G
        ngroups = (nch + SLOTS - 1) // SLOTS

        iota = lax.broadcasted_iota(jnp.int32, (L,), 0)
        ones_i = jnp.ones((L,), jnp.int32)
        zero_i = jnp.zeros((L,), jnp.int32)
        not_lane0 = iota >= 1

        def rbase(k):
            return jnp.minimum(a + k * G, emax)

        def scatter_wait(b):
            pltpu.make_async_copy(gbuf.at[b], acc.at[idxbuf.at[b]], sem_s[b]).wait()

        for b in range(SLOTS):
            @pl.when(b < nch)
            def _(b=b):
                pltpu.async_copy(col_hbm.at[pl.ds(rbase(b), G)], colbuf.at[b],
                                 sem_c[b])

        for i in range(rpt // L):
            pltpu.make_async_copy(zbuf, acc.at[pl.ds(racc, L)], sem_o).wait()

        def group(p, carry):
            ks = [SLOTS * p + b for b in range(SLOTS)]
            for b in range(SLOTS):
                k = ks[b]

                @pl.when(k >= SLOTS)
                def _():
                    scatter_wait(b)

                @pl.when(k < nch)
                def _(k=k, b=b):
                    pltpu.make_async_copy(col_hbm.at[pl.ds(rbase(k), G)],
                                          colbuf.at[b], sem_c[b]).wait()
                    pltpu.async_copy(x_hbm.at[colbuf.at[b]], gbuf.at[b], sem_g[b])

            cy = carry
            for b in range(SLOTS):
                k = ks[b]
                nominal = a + k * G
                ebase = jnp.minimum(nominal, emax)
                shift = nominal - ebase
                for v in range(nv_g):
                    cnt[pl.ds(v * L, L)] = zero_i

                def scan(state):
                    v, _ = state
                    pos = rp_t[pl.ds(v * L, L)] - ebase
                    msk = ((pos >= shift) & (pos < G) & ((iota + v * L) >= 1))
                    plsc.addupdate_scatter(cnt, [pos], ones_i, mask=msk)
                    return v + 1, (pos[L - 1] < G) & (v + 1 < nv_rp)

                v0 = jnp.minimum((cy + 1) // L, nv_rp - 1)
                lax.while_loop(lambda s: s[1], scan, (v0, jnp.bool_(True)))
                cyb = cy
                for v in range(nv_g):
                    run = plsc.cumsum(cnt[pl.ds(v * L, L)]) + cyb
                    epos = iota + (ebase + v * L)
                    keep = (epos >= rp0) & (epos >= nominal) & (epos < rend)
                    idxbuf[b, pl.ds(v * L, L)] = jnp.where(keep, racc + run, trash)
                    cyb = run[L - 1]
                cy = jnp.where(k < nch, cyb, cy)

            for b in range(SLOTS):
                k = ks[b]

                @pl.when(k < nch)
                def _(k=k, b=b):
                    pltpu.make_async_copy(x_hbm.at[colbuf.at[b]], gbuf.at[b],
                                          sem_g[b]).wait()

                    @pl.when(k + SLOTS < nch)
                    def _():
                        pltpu.async_copy(col_hbm.at[pl.ds(rbase(k + SLOTS), G)],
                                         colbuf.at[b], sem_c[b])

                    pltpu.async_copy(gbuf.at[b], acc.at[idxbuf.at[b]], sem_s[b],
                                     add=True)
            return cy

        lax.fori_loop(0, ngroups, group, jnp.int32(0))

        for b in range(SLOTS):
            j = SLOTS * (ngroups - 1) + b

            @pl.when((j >= 0) & (j < nch))
            def _(b=b):
                scatter_wait(b)

        for i in range(rpt // L):
            dst = jnp.minimum(r0 + i * L, n - L)
            pltpu.async_copy(acc.at[pl.ds(racc + (dst - r0), L)],
                             out_hbm.at[pl.ds(dst, L)], sem_o)
        for i in range(rpt // L):
            pltpu.make_async_copy(acc.at[pl.ds(racc, L)],
                                  out_hbm.at[pl.ds(r0, L)], sem_o).wait()

    return sag, rpt, rp_cols


def kernel(X, row_pointers, column_index, blockPartition, edgeToColumn,
           edgeToRow, hybrid_type, row_nzr, col_nzr):
    n, d = X.shape
    e = column_index.shape[0]
    sag, rpt, rp_cols = _build_sag(n, e, d)

    nt = NC * NS
    rp_ext = jnp.concatenate(
        [row_pointers.astype(jnp.int32),
         jnp.full((nt * rpt + rp_cols - (n + 1),), e, jnp.int32)])

    return sag(X, rp_ext, column_index)
